# baseline (speedup 1.0000x reference)
import jax
import jax.numpy as jnp
from jax.experimental import pallas as pl
from jax.experimental.pallas import tpu as pltpu


def _dice_body(smooth_ref, inp_ref, tgt_ref, out_ref, acc_ref, r_ref):
    n = pl.program_id(0)
    hb = pl.program_id(1)
    num_n = pl.num_programs(0)
    num_hb = pl.num_programs(1)
    x = inp_ref[0]
    t = tgt_ref[0]
    smooth = smooth_ref[0, 0]
    C = x.shape[0]

    @pl.when((n == 0) & (hb == 0))
    def _init_r():
        r_ref[0] = 0.0

    @pl.when(hb == 0)
    def _init_acc():
        for c in range(C):
            acc_ref[0, c] = 0.0
            acc_ref[1, c] = 0.0
            acc_ref[2, c] = 0.0

    for c in range(C):
        xc = x[c]
        m = (t == c).astype(jnp.float32)
        acc_ref[0, c] += jnp.sum(m * xc)
        acc_ref[1, c] += jnp.sum(xc)
        acc_ref[2, c] += jnp.sum(m)

    @pl.when(hb == num_hb - 1)
    def _fold_sample():
        r = jnp.float32(0.0)
        for c in range(C):
            r = r + (2.0 * acc_ref[0, c] + smooth) / (
                acc_ref[1, c] + acc_ref[2, c] + smooth)
        r_ref[0] = r_ref[0] + r

        @pl.when(n == num_n - 1)
        def _fini():
            out_ref[0, 0] = 1.0 - r_ref[0] / (num_n * C)


def kernel(inputs, targets, smooth):
    N, C, H, W = inputs.shape
    HB = 4
    BH = H // HB
    t32 = targets.astype(jnp.int32)
    s = jnp.asarray(smooth, jnp.float32).reshape(1, 1)
    out = pl.pallas_call(
        _dice_body,
        grid=(N, HB),
        in_specs=[
            pl.BlockSpec(memory_space=pltpu.SMEM),
            pl.BlockSpec((1, C, BH, W), lambda n, h: (n, 0, h, 0)),
            pl.BlockSpec((1, BH, W), lambda n, h: (n, h, 0)),
        ],
        out_specs=pl.BlockSpec(memory_space=pltpu.SMEM),
        out_shape=jax.ShapeDtypeStruct((1, 1), jnp.float32),
        scratch_shapes=[
            pltpu.SMEM((3, C), jnp.float32),
            pltpu.SMEM((1,), jnp.float32),
        ],
    )(s, inputs, t32)
    return out[0, 0]


# speedup vs baseline: 1.6137x; 1.6137x over previous
import jax
import jax.numpy as jnp
from jax.experimental import pallas as pl
from jax.experimental.pallas import tpu as pltpu


def _dice_body(smooth_ref, inp_ref, tgt_ref, out_ref, r_ref):
    n = pl.program_id(0)
    num_n = pl.num_programs(0)
    x = inp_ref[0]
    t = tgt_ref[0]
    smooth = smooth_ref[0, 0]
    C, H, W = x.shape

    ones_row = jnp.ones((1, H), jnp.float32)

    @pl.when(n == 0)
    def _init_r():
        r_ref[0] = 0.0

    r = jnp.float32(0.0)
    for c in range(C):
        xc = x[c]
        m = (t == c).astype(jnp.float32)
        inter = jnp.sum(jax.lax.dot(ones_row, m * xc))
        xsum = jnp.sum(jax.lax.dot(ones_row, xc))
        cnt = jnp.sum(jax.lax.dot(ones_row, m))
        r = r + (2.0 * inter + smooth) / (xsum + cnt + smooth)
    r_ref[0] = r_ref[0] + r

    @pl.when(n == num_n - 1)
    def _fini():
        out_ref[0, 0] = 1.0 - r_ref[0] / (num_n * C)


def kernel(inputs, targets, smooth):
    N, C, H, W = inputs.shape
    t32 = targets.astype(jnp.int32)
    s = jnp.asarray(smooth, jnp.float32).reshape(1, 1)
    out = pl.pallas_call(
        _dice_body,
        grid=(N,),
        in_specs=[
            pl.BlockSpec(memory_space=pltpu.SMEM),
            pl.BlockSpec((1, C, H, W), lambda n: (n, 0, 0, 0)),
            pl.BlockSpec((1, H, W), lambda n: (n, 0, 0)),
        ],
        out_specs=pl.BlockSpec(memory_space=pltpu.SMEM),
        out_shape=jax.ShapeDtypeStruct((1, 1), jnp.float32),
        scratch_shapes=[pltpu.SMEM((1,), jnp.float32)],
    )(s, inputs, t32)
    return out[0, 0]


# speedup vs baseline: 1.6808x; 1.0416x over previous
import jax
import jax.numpy as jnp
from jax.experimental import pallas as pl
from jax.experimental.pallas import tpu as pltpu


def _dice_body(smooth_ref, inp_ref, tgt_ref, out_ref, r_ref):
    n = pl.program_id(0)
    num_n = pl.num_programs(0)
    x = inp_ref[0]
    t = tgt_ref[0]
    smooth = smooth_ref[0, 0]
    C, H, W = x.shape

    @pl.when(n == 0)
    def _init_r():
        r_ref[0] = 0.0

    r = jnp.float32(0.0)
    for c in range(C):
        xc = x[c]
        eq = t == c
        inter = jnp.sum(jnp.where(eq, xc, 0.0))
        xsum = jnp.sum(xc)
        cnt = jnp.sum(jnp.where(eq, 1.0, 0.0))
        r = r + (2.0 * inter + smooth) / (xsum + cnt + smooth)
    r_ref[0] = r_ref[0] + r

    @pl.when(n == num_n - 1)
    def _fini():
        out_ref[0, 0] = 1.0 - r_ref[0] / (num_n * C)


def kernel(inputs, targets, smooth):
    N, C, H, W = inputs.shape
    t32 = targets.astype(jnp.int32)
    s = jnp.asarray(smooth, jnp.float32).reshape(1, 1)
    out = pl.pallas_call(
        _dice_body,
        grid=(N,),
        in_specs=[
            pl.BlockSpec(memory_space=pltpu.SMEM),
            pl.BlockSpec((1, C, H, W), lambda n: (n, 0, 0, 0)),
            pl.BlockSpec((1, H, W), lambda n: (n, 0, 0)),
        ],
        out_specs=pl.BlockSpec(memory_space=pltpu.SMEM),
        out_shape=jax.ShapeDtypeStruct((1, 1), jnp.float32),
        scratch_shapes=[pltpu.SMEM((1,), jnp.float32)],
    )(s, inputs, t32)
    return out[0, 0]
